# baseline (speedup 1.0000x reference)
import functools

import jax
import jax.numpy as jnp
from jax import lax
from jax.experimental import pallas as pl
from jax.experimental.pallas import tpu as pltpu
from jax.experimental.pallas import tpu_sc as plsc

_N = 10000
_E = 320000
_F = 128
_FH = 64
_ORDER = 8

_NC = 2
_NS = 16
_L = 16

_NPAD = 10240
_RT = _NPAD // _NS
_CROWS = 64
_NCHUNK = _RT // _CROWS

_EB = 128
_BPC = 16
_NECH = 10
_EPT = _NECH * _BPC * _EB
_EPAD = _NS * _EPT
_DUMMY = _N
_NBUF = 4


def _rsqrt16(x):
    i = lax.bitcast_convert_type(x, jnp.int32)
    i = jnp.int32(0x5F3759DF) - (i >> 1)
    y = lax.bitcast_convert_type(i, jnp.float32)
    for _ in range(3):
        y = y * (1.5 - 0.5 * x * y * y)
    return y


def _grand_body(xp, rows_hbm, grows_hbm, cols_hbm, y_hbm, w_hbm,
                acc_sh, deg_sh,
                rows_cv, cols_cv, bufs, abuf, wbuf, ybuf, zbuf,
                onesb, dbuf, dis_t, semG, semS):
    cid = lax.axis_index("c")
    sid = lax.axis_index("s")
    rbase = sid * _RT
    wrow0 = cid * _NPAD
    zero16 = jnp.zeros((_L,), jnp.float32)
    one16 = zero16 + 1.0

    def _fill(i, _):
        for k in range(_FH // _L):
            zbuf[i, pl.ds(k * _L, _L)] = zero16
        return 0
    lax.fori_loop(0, _CROWS, _fill, 0)

    def _fillo(i, _):
        onesb[i, pl.ds(0, _L)] = one16
        return 0
    lax.fori_loop(0, _EB, _fillo, 0)

    def _zacc(c, _):
        pltpu.sync_copy(zbuf, acc_sh.at[pl.ds(rbase + c * _CROWS, _CROWS)])
        return 0
    lax.fori_loop(0, _NCHUNK, _zacc, 0)
    def _zdeg(c, _):
        pltpu.sync_copy(zbuf.at[pl.ds(0, _CROWS), pl.ds(0, _L)],
                        deg_sh.at[pl.ds(rbase + c * _CROWS, _CROWS)])
        return 0
    lax.fori_loop(0, _NCHUNK, _zdeg, 0)
    plsc.subcore_barrier()

    def _deg_chunk(ch, _):
        pltpu.sync_copy(rows_hbm.at[sid, ch], rows_cv)
        def _deg(b, _2):
            pltpu.async_copy(onesb, deg_sh.at[rows_cv.at[b]], semS[0], add=True)
            return 0
        lax.fori_loop(0, _BPC, _deg, 0)
        def _degw(b, _2):
            pltpu.make_async_copy(onesb, deg_sh.at[rows_cv.at[b]], semS[0]).wait()
            return 0
        lax.fori_loop(0, _BPC, _degw, 0)
        return 0
    lax.fori_loop(0, _NECH, _deg_chunk, 0)
    plsc.subcore_barrier()

    def _dis_chunk(c, _):
        pltpu.sync_copy(deg_sh.at[pl.ds(rbase + c * _CROWS, _CROWS)], dbuf)
        def _dis_row(i, _2):
            degv = dbuf[i, pl.ds(0, _L)] + 1.0
            dis_t[c * _CROWS + i, pl.ds(0, _L)] = _rsqrt16(degv)
            return 0
        lax.fori_loop(0, _CROWS, _dis_row, 0)
        return 0
    lax.fori_loop(0, _NCHUNK, _dis_chunk, 0)

    def _init_chunk(c, _):
        g0 = rbase + c * _CROWS
        pltpu.sync_copy(xp.at[cid, pl.ds(g0, _CROWS)], abuf)
        def _init_row(i, _2):
            d = dis_t[c * _CROWS + i, pl.ds(0, _L)]
            for k in range(_FH // _L):
                xv = abuf[i, pl.ds(k * _L, _L)]
                wbuf[i, pl.ds(k * _L, _L)] = (0.5 * xv) * d
                ybuf[i, pl.ds(k * _L, _L)] = 0.5 * xv
            return 0
        lax.fori_loop(0, _CROWS, _init_row, 0)
        pltpu.sync_copy(wbuf, w_hbm.at[pl.ds(wrow0 + g0, _CROWS)])
        pltpu.sync_copy(ybuf, y_hbm.at[cid, pl.ds(g0, _CROWS)])
        return 0
    lax.fori_loop(0, _NCHUNK, _init_chunk, 0)
    plsc.subcore_barrier()

    def _round(t, _):
        def _echunk(ch, _2):
            pltpu.sync_copy(grows_hbm.at[cid, sid, ch], rows_cv)
            pltpu.sync_copy(cols_hbm.at[sid, ch], cols_cv)
            for b in range(_NBUF):
                pltpu.async_copy(w_hbm.at[rows_cv.at[b]], bufs[b], semG[b])
            def _equad(q, _3):
                for b in range(_NBUF):
                    blk = q * _NBUF + b
                    pltpu.make_async_copy(
                        w_hbm.at[rows_cv.at[blk]], bufs[b], semG[b]).wait()
                    pltpu.async_copy(
                        bufs[b], acc_sh.at[cols_cv.at[blk]], semS[b], add=True)
                @pl.when(q < _BPC // _NBUF - 1)
                def _issue_next():
                    for b in range(_NBUF):
                        blk = q * _NBUF + b
                        pltpu.make_async_copy(
                            bufs[b], acc_sh.at[cols_cv.at[blk]], semS[b]).wait()
                        pltpu.async_copy(
                            w_hbm.at[rows_cv.at[blk + _NBUF]], bufs[b], semG[b])
                return 0
            lax.fori_loop(0, _BPC // _NBUF, _equad, 0)
            for b in range(_NBUF):
                pltpu.make_async_copy(
                    bufs[b], acc_sh.at[cols_cv.at[_BPC - _NBUF + b]],
                    semS[b]).wait()
            return 0
        lax.fori_loop(0, _NECH, _echunk, 0)
        plsc.subcore_barrier()

        def _row_chunk(c, _2):
            g0 = rbase + c * _CROWS
            pltpu.sync_copy(acc_sh.at[pl.ds(g0, _CROWS)], abuf)
            pltpu.sync_copy(w_hbm.at[pl.ds(wrow0 + g0, _CROWS)], wbuf)
            pltpu.sync_copy(y_hbm.at[cid, pl.ds(g0, _CROWS)], ybuf)
            def _row(i, _3):
                d = dis_t[c * _CROWS + i, pl.ds(0, _L)]
                d2 = d * d
                for k in range(_FH // _L):
                    g = (abuf[i, pl.ds(k * _L, _L)]
                         + wbuf[i, pl.ds(k * _L, _L)])
                    ybuf[i, pl.ds(k * _L, _L)] += d * g
                    wbuf[i, pl.ds(k * _L, _L)] = d2 * g
                return 0
            lax.fori_loop(0, _CROWS, _row, 0)
            pltpu.sync_copy(wbuf, w_hbm.at[pl.ds(wrow0 + g0, _CROWS)])
            pltpu.sync_copy(ybuf, y_hbm.at[cid, pl.ds(g0, _CROWS)])
            pltpu.sync_copy(zbuf, acc_sh.at[pl.ds(g0, _CROWS)])
            return 0
        lax.fori_loop(0, _NCHUNK, _row_chunk, 0)
        plsc.subcore_barrier()
        return 0
    lax.fori_loop(0, _ORDER, _round, 0)


_grand_sc = functools.partial(
    pl.kernel,
    out_type=(
        jax.ShapeDtypeStruct((_NC, _NPAD, _FH), jnp.float32),
        jax.ShapeDtypeStruct((_NC * _NPAD, _FH), jnp.float32),
    ),
    mesh=plsc.VectorSubcoreMesh(core_axis_name="c", subcore_axis_name="s",
                                num_cores=_NC, num_subcores=_NS),
    compiler_params=pltpu.CompilerParams(use_tc_tiling_on_sc=False),
    scratch_types=[
        pltpu.VMEM_SHARED((_NPAD, _FH), jnp.float32),
        pltpu.VMEM_SHARED((_NPAD, _L), jnp.float32),
        pltpu.VMEM((_BPC, _EB), jnp.int32),
        pltpu.VMEM((_BPC, _EB), jnp.int32),
        [pltpu.VMEM((_EB, _FH), jnp.float32)] * _NBUF,
        pltpu.VMEM((_CROWS, _FH), jnp.float32),
        pltpu.VMEM((_CROWS, _FH), jnp.float32),
        pltpu.VMEM((_CROWS, _FH), jnp.float32),
        pltpu.VMEM((_CROWS, _FH), jnp.float32),
        pltpu.VMEM((_EB, _L), jnp.float32),
        pltpu.VMEM((_CROWS, _L), jnp.float32),
        pltpu.VMEM((_RT, _L), jnp.float32),
        [pltpu.SemaphoreType.DMA] * _NBUF,
        [pltpu.SemaphoreType.DMA] * _NBUF,
    ],
)(_grand_body)


def _mlp_body(y0_ref, y1_ref, w1a_ref, w1b_ref, b1_ref, w2_ref, b2_ref, o_ref):
    acc = jnp.dot(y0_ref[...], w1a_ref[...], preferred_element_type=jnp.float32)
    acc += jnp.dot(y1_ref[...], w1b_ref[...], preferred_element_type=jnp.float32)
    h = jnp.maximum(acc * (1.0 / (_ORDER + 1.0)) + b1_ref[...], 0.0)
    o_ref[...] = jnp.dot(h, w2_ref[...], preferred_element_type=jnp.float32) + b2_ref[...]


def _mlp(y0, y1, w1a, w1b, b1, w2, b2):
    blk = 2000
    grid = _N // blk
    return pl.pallas_call(
        _mlp_body,
        grid=(grid,),
        in_specs=[
            pl.BlockSpec((blk, _FH), lambda i: (i, 0)),
            pl.BlockSpec((blk, _FH), lambda i: (i, 0)),
            pl.BlockSpec((_FH, 64), lambda i: (0, 0)),
            pl.BlockSpec((_FH, 64), lambda i: (0, 0)),
            pl.BlockSpec((1, 64), lambda i: (0, 0)),
            pl.BlockSpec((64, 32), lambda i: (0, 0)),
            pl.BlockSpec((1, 32), lambda i: (0, 0)),
        ],
        out_specs=pl.BlockSpec((blk, 32), lambda i: (i, 0)),
        out_shape=jax.ShapeDtypeStruct((_N, 32), jnp.float32),
    )(y0, y1, w1a, w1b, b1, w2, b2)


def kernel(x, edge_index, W1, b1, W2, b2):
    rows = edge_index[0]
    cols = edge_index[1]
    pad = _EPAD - _E
    fake = jnp.full((pad,), _DUMMY, dtype=jnp.int32)
    rows_p = jnp.concatenate([rows, fake]).reshape(_NS, _NECH, _BPC, _EB)
    cols_p = jnp.concatenate([cols, fake]).reshape(_NS, _NECH, _BPC, _EB)
    grows = (rows_p[None] +
             (jnp.arange(_NC, dtype=jnp.int32) * _NPAD)[:, None, None, None, None])
    xp = jnp.pad(x, ((0, _NPAD - _N), (0, 0)))
    xp = xp.reshape(_NPAD, _NC, _FH).transpose(1, 0, 2)
    y2, _ = _grand_sc(xp, rows_p, grows, cols_p)
    return _mlp(y2[0, :_N], y2[1, :_N],
                W1[:_FH], W1[_FH:], b1.reshape(1, -1), W2, b2.reshape(1, -1))


# speedup vs baseline: 1.6872x; 1.6872x over previous
import functools

import jax
import jax.numpy as jnp
from jax import lax
from jax.experimental import pallas as pl
from jax.experimental.pallas import tpu as pltpu
from jax.experimental.pallas import tpu_sc as plsc

_N = 10000
_E = 320000
_F = 128
_FH = 64
_ORDER = 8

_NC = 2
_NS = 16
_L = 16

_NPAD = 10240
_RT = _NPAD // _NS
_CROWS = 32
_NCHUNK = _RT // _CROWS

_EB = 128
_BPC = 8
_NECH = 20
_EPT = _NECH * _BPC * _EB
_EPAD = _NS * _EPT
_DUMMY = _N


def _rsqrt16(x):
    i = lax.bitcast_convert_type(x, jnp.int32)
    i = jnp.int32(0x5F3759DF) - (i >> 1)
    y = lax.bitcast_convert_type(i, jnp.float32)
    for _ in range(3):
        y = y * (1.5 - 0.5 * x * y * y)
    return y


def _grand_body(xp, rows_hbm, cols_hbm, y_hbm,
                w_sh, acc_sh, deg_sh,
                rows_cv, cols_cv, bufA, bufB, abuf, wbuf, ybuf,
                onesb, dbuf, dis_t, semG0, semG1, semS0, semS1):
    cid = lax.axis_index("c")
    sid = lax.axis_index("s")
    rbase = sid * _RT
    zero16 = jnp.zeros((_L,), jnp.float32)
    one16 = zero16 + 1.0

    def _fill(i, _):
        for k in range(_FH // _L):
            wbuf[i, pl.ds(k * _L, _L)] = zero16
        return 0
    lax.fori_loop(0, _CROWS, _fill, 0)

    def _fillo(i, _):
        onesb[i, pl.ds(0, _L)] = one16
        return 0
    lax.fori_loop(0, _EB, _fillo, 0)

    def _zacc(c, _):
        pltpu.sync_copy(wbuf, acc_sh.at[pl.ds(rbase + c * _CROWS, _CROWS)])
        pltpu.sync_copy(wbuf.at[pl.ds(0, _CROWS), pl.ds(0, _L)],
                        deg_sh.at[pl.ds(rbase + c * _CROWS, _CROWS)])
        return 0
    lax.fori_loop(0, _NCHUNK, _zacc, 0)
    plsc.subcore_barrier()

    def _deg_chunk(ch, _):
        pltpu.sync_copy(rows_hbm.at[sid, ch], rows_cv)
        def _deg(b, _2):
            pltpu.async_copy(onesb, deg_sh.at[rows_cv.at[b]], semS0, add=True)
            return 0
        lax.fori_loop(0, _BPC, _deg, 0)
        def _degw(b, _2):
            pltpu.make_async_copy(onesb, deg_sh.at[rows_cv.at[b]], semS0).wait()
            return 0
        lax.fori_loop(0, _BPC, _degw, 0)
        return 0
    lax.fori_loop(0, _NECH, _deg_chunk, 0)
    plsc.subcore_barrier()

    def _dis_chunk(c, _):
        pltpu.sync_copy(deg_sh.at[pl.ds(rbase + c * _CROWS, _CROWS)], dbuf)
        def _dis_row(i, _2):
            degv = dbuf[i, pl.ds(0, _L)] + 1.0
            dis_t[c * _CROWS + i, pl.ds(0, _L)] = _rsqrt16(degv)
            return 0
        lax.fori_loop(0, _CROWS, _dis_row, 0)
        return 0
    lax.fori_loop(0, _NCHUNK, _dis_chunk, 0)

    def _init_chunk(c, _):
        g0 = rbase + c * _CROWS
        pltpu.sync_copy(xp.at[cid, pl.ds(g0, _CROWS)], abuf)
        def _init_row(i, _2):
            d = dis_t[c * _CROWS + i, pl.ds(0, _L)]
            for k in range(_FH // _L):
                xv = abuf[i, pl.ds(k * _L, _L)]
                wbuf[i, pl.ds(k * _L, _L)] = (0.5 * xv) * d
                ybuf[i, pl.ds(k * _L, _L)] = 0.5 * xv
            return 0
        lax.fori_loop(0, _CROWS, _init_row, 0)
        pltpu.sync_copy(wbuf, w_sh.at[pl.ds(g0, _CROWS)])
        pltpu.sync_copy(ybuf, y_hbm.at[cid, pl.ds(g0, _CROWS)])
        return 0
    lax.fori_loop(0, _NCHUNK, _init_chunk, 0)
    plsc.subcore_barrier()

    def _round(t, _):
        def _echunk(ch, _2):
            pltpu.sync_copy(rows_hbm.at[sid, ch], rows_cv)
            pltpu.sync_copy(cols_hbm.at[sid, ch], cols_cv)
            pltpu.async_copy(w_sh.at[rows_cv.at[0]], bufA, semG0)
            pltpu.async_copy(w_sh.at[rows_cv.at[1]], bufB, semG1)
            def _epair(p, _3):
                b0 = 2 * p
                b1 = 2 * p + 1
                pltpu.make_async_copy(w_sh.at[rows_cv.at[b0]], bufA, semG0).wait()
                pltpu.async_copy(bufA, acc_sh.at[cols_cv.at[b0]], semS0, add=True)
                pltpu.make_async_copy(w_sh.at[rows_cv.at[b1]], bufB, semG1).wait()
                pltpu.async_copy(bufB, acc_sh.at[cols_cv.at[b1]], semS1, add=True)
                @pl.when(p < _BPC // 2 - 1)
                def _issue_next():
                    pltpu.make_async_copy(bufA, acc_sh.at[cols_cv.at[b0]], semS0).wait()
                    pltpu.async_copy(w_sh.at[rows_cv.at[b0 + 2]], bufA, semG0)
                    pltpu.make_async_copy(bufB, acc_sh.at[cols_cv.at[b1]], semS1).wait()
                    pltpu.async_copy(w_sh.at[rows_cv.at[b1 + 2]], bufB, semG1)
                return 0
            lax.fori_loop(0, _BPC // 2, _epair, 0)
            pltpu.make_async_copy(bufA, acc_sh.at[cols_cv.at[_BPC - 2]], semS0).wait()
            pltpu.make_async_copy(bufB, acc_sh.at[cols_cv.at[_BPC - 1]], semS1).wait()
            return 0
        lax.fori_loop(0, _NECH, _echunk, 0)
        plsc.subcore_barrier()

        def _row_chunk(c, _2):
            g0 = rbase + c * _CROWS
            pltpu.sync_copy(acc_sh.at[pl.ds(g0, _CROWS)], abuf)
            pltpu.sync_copy(w_sh.at[pl.ds(g0, _CROWS)], wbuf)
            pltpu.sync_copy(y_hbm.at[cid, pl.ds(g0, _CROWS)], ybuf)
            def _row(i, _3):
                d = dis_t[c * _CROWS + i, pl.ds(0, _L)]
                d2 = d * d
                for k in range(_FH // _L):
                    g = (abuf[i, pl.ds(k * _L, _L)]
                         + wbuf[i, pl.ds(k * _L, _L)])
                    ybuf[i, pl.ds(k * _L, _L)] += d * g
                    wbuf[i, pl.ds(k * _L, _L)] = d2 * g
                    abuf[i, pl.ds(k * _L, _L)] = zero16
                return 0
            lax.fori_loop(0, _CROWS, _row, 0)
            pltpu.sync_copy(wbuf, w_sh.at[pl.ds(g0, _CROWS)])
            pltpu.sync_copy(ybuf, y_hbm.at[cid, pl.ds(g0, _CROWS)])
            pltpu.sync_copy(abuf, acc_sh.at[pl.ds(g0, _CROWS)])
            return 0
        lax.fori_loop(0, _NCHUNK, _row_chunk, 0)
        plsc.subcore_barrier()
        return 0
    lax.fori_loop(0, _ORDER, _round, 0)


_grand_sc = functools.partial(
    pl.kernel,
    out_type=jax.ShapeDtypeStruct((_NC, _NPAD, _FH), jnp.float32),
    mesh=plsc.VectorSubcoreMesh(core_axis_name="c", subcore_axis_name="s",
                                num_cores=_NC, num_subcores=_NS),
    compiler_params=pltpu.CompilerParams(use_tc_tiling_on_sc=False),
    scratch_types=[
        pltpu.VMEM_SHARED((_NPAD, _FH), jnp.float32),
        pltpu.VMEM_SHARED((_NPAD, _FH), jnp.float32),
        pltpu.VMEM_SHARED((_NPAD, _L), jnp.float32),
        pltpu.VMEM((_BPC, _EB), jnp.int32),
        pltpu.VMEM((_BPC, _EB), jnp.int32),
        pltpu.VMEM((_EB, _FH), jnp.float32),
        pltpu.VMEM((_EB, _FH), jnp.float32),
        pltpu.VMEM((_CROWS, _FH), jnp.float32),
        pltpu.VMEM((_CROWS, _FH), jnp.float32),
        pltpu.VMEM((_CROWS, _FH), jnp.float32),
        pltpu.VMEM((_EB, _L), jnp.float32),
        pltpu.VMEM((_CROWS, _L), jnp.float32),
        pltpu.VMEM((_RT, _L), jnp.float32),
        pltpu.SemaphoreType.DMA,
        pltpu.SemaphoreType.DMA,
        pltpu.SemaphoreType.DMA,
        pltpu.SemaphoreType.DMA,
    ],
)(_grand_body)


def _mlp_body(y0_ref, y1_ref, w1a_ref, w1b_ref, b1_ref, w2_ref, b2_ref, o_ref):
    acc = jnp.dot(y0_ref[...], w1a_ref[...], preferred_element_type=jnp.float32)
    acc += jnp.dot(y1_ref[...], w1b_ref[...], preferred_element_type=jnp.float32)
    h = jnp.maximum(acc * (1.0 / (_ORDER + 1.0)) + b1_ref[...], 0.0)
    o_ref[...] = jnp.dot(h, w2_ref[...], preferred_element_type=jnp.float32) + b2_ref[...]


def _mlp(y0, y1, w1a, w1b, b1, w2, b2):
    blk = 2000
    grid = _N // blk
    return pl.pallas_call(
        _mlp_body,
        grid=(grid,),
        in_specs=[
            pl.BlockSpec((blk, _FH), lambda i: (i, 0)),
            pl.BlockSpec((blk, _FH), lambda i: (i, 0)),
            pl.BlockSpec((_FH, 64), lambda i: (0, 0)),
            pl.BlockSpec((_FH, 64), lambda i: (0, 0)),
            pl.BlockSpec((1, 64), lambda i: (0, 0)),
            pl.BlockSpec((64, 32), lambda i: (0, 0)),
            pl.BlockSpec((1, 32), lambda i: (0, 0)),
        ],
        out_specs=pl.BlockSpec((blk, 32), lambda i: (i, 0)),
        out_shape=jax.ShapeDtypeStruct((_N, 32), jnp.float32),
    )(y0, y1, w1a, w1b, b1, w2, b2)


def kernel(x, edge_index, W1, b1, W2, b2):
    rows = edge_index[0]
    cols = edge_index[1]
    pad = _EPAD - _E
    fake = jnp.full((pad,), _DUMMY, dtype=jnp.int32)
    rows_p = jnp.concatenate([rows, fake]).reshape(_NS, _NECH, _BPC, _EB)
    cols_p = jnp.concatenate([cols, fake]).reshape(_NS, _NECH, _BPC, _EB)
    xp = jnp.pad(x, ((0, _NPAD - _N), (0, 0)))
    xp = xp.reshape(_NPAD, _NC, _FH).transpose(1, 0, 2)
    y2 = _grand_sc(xp, rows_p, cols_p)
    return _mlp(y2[0, :_N], y2[1, :_N],
                W1[:_FH], W1[_FH:], b1.reshape(1, -1), W2, b2.reshape(1, -1))


# speedup vs baseline: 1.7049x; 1.0105x over previous
import functools

import jax
import jax.numpy as jnp
from jax import lax
from jax.experimental import pallas as pl
from jax.experimental.pallas import tpu as pltpu
from jax.experimental.pallas import tpu_sc as plsc

_N = 10000
_E = 320000
_F = 128
_FH = 64
_ORDER = 8

_NC = 2
_NS = 16
_L = 16

_NPAD = 10240
_RT = _NPAD // _NS
_CROWS = 32
_NCHUNK = _RT // _CROWS

_EB = 128
_BPC = 8
_NECH = 20
_EPT = _NECH * _BPC * _EB
_EPAD = _NS * _EPT
_DUMMY = _N


def _rsqrt16(x):
    i = lax.bitcast_convert_type(x, jnp.int32)
    i = jnp.int32(0x5F3759DF) - (i >> 1)
    y = lax.bitcast_convert_type(i, jnp.float32)
    for _ in range(3):
        y = y * (1.5 - 0.5 * x * y * y)
    return y


def _grand_body(xp, rows_hbm, cols_hbm, y_hbm,
                w_sh, acc_sh, deg_sh,
                rows_cv, cols_cv, bufA, bufB, abuf, wbuf, ybuf,
                onesb, dbuf, dis_t, semG0, semG1, semS0, semS1):
    cid = lax.axis_index("c")
    sid = lax.axis_index("s")
    rbase = sid * _RT
    zero16 = jnp.zeros((_L,), jnp.float32)
    one16 = zero16 + 1.0

    def _fill(i, _):
        for k in range(_FH // _L):
            wbuf[i, pl.ds(k * _L, _L)] = zero16
        return 0
    lax.fori_loop(0, _CROWS, _fill, 0)

    def _fillo(i, _):
        onesb[i, pl.ds(0, _L)] = one16
        return 0
    lax.fori_loop(0, _EB, _fillo, 0)

    def _zacc(c, _):
        pltpu.sync_copy(wbuf, acc_sh.at[pl.ds(rbase + c * _CROWS, _CROWS)])
        pltpu.sync_copy(wbuf.at[pl.ds(0, _CROWS), pl.ds(0, _L)],
                        deg_sh.at[pl.ds(rbase + c * _CROWS, _CROWS)])
        return 0
    lax.fori_loop(0, _NCHUNK, _zacc, 0)
    plsc.subcore_barrier()

    def _deg_chunk(ch, _):
        pltpu.sync_copy(rows_hbm.at[sid, ch], rows_cv)
        def _deg(b, _2):
            pltpu.async_copy(onesb, deg_sh.at[rows_cv.at[b]], semS0, add=True)
            return 0
        lax.fori_loop(0, _BPC, _deg, 0)
        def _degw(b, _2):
            pltpu.make_async_copy(onesb, deg_sh.at[rows_cv.at[b]], semS0).wait()
            return 0
        lax.fori_loop(0, _BPC, _degw, 0)
        return 0
    lax.fori_loop(0, _NECH, _deg_chunk, 0)
    plsc.subcore_barrier()

    def _dis_chunk(c, _):
        pltpu.sync_copy(deg_sh.at[pl.ds(rbase + c * _CROWS, _CROWS)], dbuf)
        def _dis_row(i, _2):
            degv = dbuf[i, pl.ds(0, _L)] + 1.0
            dis_t[c * _CROWS + i, pl.ds(0, _L)] = _rsqrt16(degv)
            return 0
        lax.fori_loop(0, _CROWS, _dis_row, 0)
        return 0
    lax.fori_loop(0, _NCHUNK, _dis_chunk, 0)

    def _init_chunk(c, _):
        g0 = rbase + c * _CROWS
        pltpu.sync_copy(xp.at[pl.ds(g0, _CROWS), pl.ds(cid * _FH, _FH)], abuf)
        def _init_row(i, _2):
            d = dis_t[c * _CROWS + i, pl.ds(0, _L)]
            for k in range(_FH // _L):
                xv = abuf[i, pl.ds(k * _L, _L)]
                wbuf[i, pl.ds(k * _L, _L)] = (0.5 * xv) * d
                ybuf[i, pl.ds(k * _L, _L)] = 0.5 * xv
            return 0
        lax.fori_loop(0, _CROWS, _init_row, 0)
        pltpu.sync_copy(wbuf, w_sh.at[pl.ds(g0, _CROWS)])
        pltpu.sync_copy(ybuf, y_hbm.at[cid, pl.ds(g0, _CROWS)])
        return 0
    lax.fori_loop(0, _NCHUNK, _init_chunk, 0)
    plsc.subcore_barrier()

    def _round(t, _):
        def _echunk(ch, _2):
            pltpu.sync_copy(rows_hbm.at[sid, ch], rows_cv)
            pltpu.sync_copy(cols_hbm.at[sid, ch], cols_cv)
            pltpu.async_copy(w_sh.at[rows_cv.at[0]], bufA, semG0)
            pltpu.async_copy(w_sh.at[rows_cv.at[1]], bufB, semG1)
            def _epair(p, _3):
                b0 = 2 * p
                b1 = 2 * p + 1
                pltpu.make_async_copy(w_sh.at[rows_cv.at[b0]], bufA, semG0).wait()
                pltpu.async_copy(bufA, acc_sh.at[cols_cv.at[b0]], semS0, add=True)
                pltpu.make_async_copy(w_sh.at[rows_cv.at[b1]], bufB, semG1).wait()
                pltpu.async_copy(bufB, acc_sh.at[cols_cv.at[b1]], semS1, add=True)
                @pl.when(p < _BPC // 2 - 1)
                def _issue_next():
                    pltpu.make_async_copy(bufA, acc_sh.at[cols_cv.at[b0]], semS0).wait()
                    pltpu.async_copy(w_sh.at[rows_cv.at[b0 + 2]], bufA, semG0)
                    pltpu.make_async_copy(bufB, acc_sh.at[cols_cv.at[b1]], semS1).wait()
                    pltpu.async_copy(w_sh.at[rows_cv.at[b1 + 2]], bufB, semG1)
                return 0
            lax.fori_loop(0, _BPC // 2, _epair, 0)
            pltpu.make_async_copy(bufA, acc_sh.at[cols_cv.at[_BPC - 2]], semS0).wait()
            pltpu.make_async_copy(bufB, acc_sh.at[cols_cv.at[_BPC - 1]], semS1).wait()
            return 0
        lax.fori_loop(0, _NECH, _echunk, 0)
        plsc.subcore_barrier()

        def _row_chunk(c, _2):
            g0 = rbase + c * _CROWS
            pltpu.sync_copy(acc_sh.at[pl.ds(g0, _CROWS)], abuf)
            pltpu.sync_copy(w_sh.at[pl.ds(g0, _CROWS)], wbuf)
            pltpu.sync_copy(y_hbm.at[cid, pl.ds(g0, _CROWS)], ybuf)
            def _row(i, _3):
                d = dis_t[c * _CROWS + i, pl.ds(0, _L)]
                d2 = d * d
                for k in range(_FH // _L):
                    g = (abuf[i, pl.ds(k * _L, _L)]
                         + wbuf[i, pl.ds(k * _L, _L)])
                    ybuf[i, pl.ds(k * _L, _L)] += d * g
                    wbuf[i, pl.ds(k * _L, _L)] = d2 * g
                    abuf[i, pl.ds(k * _L, _L)] = zero16
                return 0
            lax.fori_loop(0, _CROWS, _row, 0)
            pltpu.sync_copy(wbuf, w_sh.at[pl.ds(g0, _CROWS)])
            pltpu.sync_copy(ybuf, y_hbm.at[cid, pl.ds(g0, _CROWS)])
            pltpu.sync_copy(abuf, acc_sh.at[pl.ds(g0, _CROWS)])
            return 0
        lax.fori_loop(0, _NCHUNK, _row_chunk, 0)
        plsc.subcore_barrier()
        return 0
    lax.fori_loop(0, _ORDER, _round, 0)


_grand_sc = functools.partial(
    pl.kernel,
    out_type=jax.ShapeDtypeStruct((_NC, _NPAD, _FH), jnp.float32),
    mesh=plsc.VectorSubcoreMesh(core_axis_name="c", subcore_axis_name="s",
                                num_cores=_NC, num_subcores=_NS),
    compiler_params=pltpu.CompilerParams(use_tc_tiling_on_sc=False),
    scratch_types=[
        pltpu.VMEM_SHARED((_NPAD, _FH), jnp.float32),
        pltpu.VMEM_SHARED((_NPAD, _FH), jnp.float32),
        pltpu.VMEM_SHARED((_NPAD, _L), jnp.float32),
        pltpu.VMEM((_BPC, _EB), jnp.int32),
        pltpu.VMEM((_BPC, _EB), jnp.int32),
        pltpu.VMEM((_EB, _FH), jnp.float32),
        pltpu.VMEM((_EB, _FH), jnp.float32),
        pltpu.VMEM((_CROWS, _FH), jnp.float32),
        pltpu.VMEM((_CROWS, _FH), jnp.float32),
        pltpu.VMEM((_CROWS, _FH), jnp.float32),
        pltpu.VMEM((_EB, _L), jnp.float32),
        pltpu.VMEM((_CROWS, _L), jnp.float32),
        pltpu.VMEM((_RT, _L), jnp.float32),
        pltpu.SemaphoreType.DMA,
        pltpu.SemaphoreType.DMA,
        pltpu.SemaphoreType.DMA,
        pltpu.SemaphoreType.DMA,
    ],
)(_grand_body)


def _mlp_body(y2_ref, w1a_ref, w1b_ref, b1_ref, w2_ref, b2_ref, o_ref):
    acc = jnp.dot(y2_ref[0], w1a_ref[...], preferred_element_type=jnp.float32)
    acc += jnp.dot(y2_ref[1], w1b_ref[...], preferred_element_type=jnp.float32)
    h = jnp.maximum(acc * (1.0 / (_ORDER + 1.0)) + b1_ref[...], 0.0)
    o_ref[...] = jnp.dot(h, w2_ref[...], preferred_element_type=jnp.float32) + b2_ref[...]


def _mlp(y2, w1a, w1b, b1, w2, b2):
    blk = 2000
    grid = _N // blk
    return pl.pallas_call(
        _mlp_body,
        grid=(grid,),
        in_specs=[
            pl.BlockSpec((_NC, blk, _FH), lambda i: (0, i, 0)),
            pl.BlockSpec((_FH, 64), lambda i: (0, 0)),
            pl.BlockSpec((_FH, 64), lambda i: (0, 0)),
            pl.BlockSpec((1, 64), lambda i: (0, 0)),
            pl.BlockSpec((64, 32), lambda i: (0, 0)),
            pl.BlockSpec((1, 32), lambda i: (0, 0)),
        ],
        out_specs=pl.BlockSpec((blk, 32), lambda i: (i, 0)),
        out_shape=jax.ShapeDtypeStruct((_N, 32), jnp.float32),
    )(y2, w1a, w1b, b1, w2, b2)


def kernel(x, edge_index, W1, b1, W2, b2):
    rows = edge_index[0]
    cols = edge_index[1]
    pad = _EPAD - _E
    fake = jnp.full((pad,), _DUMMY, dtype=jnp.int32)
    rows_p = jnp.concatenate([rows, fake]).reshape(_NS, _NECH, _BPC, _EB)
    cols_p = jnp.concatenate([cols, fake]).reshape(_NS, _NECH, _BPC, _EB)
    xp = jnp.pad(x, ((0, _NPAD - _N), (0, 0)))
    y2 = _grand_sc(xp, rows_p, cols_p)
    return _mlp(y2, W1[:_FH], W1[_FH:], b1.reshape(1, -1), W2, b2.reshape(1, -1))
